# baseline (speedup 1.0000x reference)
import numpy as np
import jax
import jax.numpy as jnp
from jax import lax
from jax.experimental import pallas as pl
from jax.experimental.pallas import tpu as pltpu

T = 1.3
QHAT = 0.92
LAMDA = 0.15
KREG = 5
TOPK = 12
ROWS = 8
LW = 128

_MSK = np.zeros(TOPK, np.float32)
_MSK[KREG:] = np.float32(LAMDA)
_REGCS = np.cumsum(_MSK).astype(np.float32)

_NEG_INF = np.float32(-np.inf)


def _insert(x, a):
    out = []
    cur = x
    for k in range(len(a)):
        out.append(jnp.maximum(a[k], cur))
        cur = jnp.minimum(a[k], cur)
    return out


def _body(x_ref, u_ref, mask_ref, sizes_ref):
    n = x_ref.shape[1]
    nfull = n // LW
    tail_w = n - nfull * LW

    if tail_w:
        tail = x_ref[:, nfull * LW:n]
        pad = jnp.full((ROWS, LW - tail_w), _NEG_INF, jnp.float32)
        a0 = jnp.concatenate([tail, pad], axis=1)
    else:
        a0 = jnp.full((ROWS, LW), _NEG_INF, jnp.float32)
    neg = jnp.full((ROWS, LW), _NEG_INF, jnp.float32)
    a_init = (a0,) + (neg,) * (TOPK - 1)

    def p1(c, a):
        x_c = x_ref[:, pl.ds(c * LW, LW)]
        return tuple(_insert(x_c, list(a)))

    a = list(lax.fori_loop(0, nfull, p1, a_init))

    lane = lax.broadcasted_iota(jnp.int32, (ROWS, LW), 1)
    tops = []
    for _ in range(TOPK):
        mr = jnp.max(a[0], axis=1, keepdims=True)
        il = jnp.max(jnp.where(a[0] == mr, lane, -1), axis=1, keepdims=True)
        sel = lane == il
        for k in range(TOPK - 1):
            a[k] = jnp.where(sel, a[k + 1], a[k])
        a[TOPK - 1] = jnp.where(sel, _NEG_INF, a[TOPK - 1])
        tops.append(mr)

    m_y = tops[0] / np.float32(T)

    def p2(c, acc):
        x_c = x_ref[:, pl.ds(c * LW, LW)]
        return acc + jnp.exp(x_c / np.float32(T) - m_y)

    acc0 = jnp.exp(a0 / np.float32(T) - m_y)
    acc = lax.fori_loop(0, nfull, p2, acc0)
    z = jnp.sum(acc, axis=1, keepdims=True)

    s = [jnp.exp(t / np.float32(T) - m_y) / z for t in tops]
    cs = [s[0]]
    for k in range(1, TOPK):
        cs.append(cs[-1] + s[k])
    ord_reg = [s[k] + (np.float32(LAMDA) if k >= KREG else np.float32(0.0))
               for k in range(TOPK)]
    cs_reg = [cs[k] + _REGCS[k] for k in range(TOPK)]

    cnt = jnp.zeros_like(tops[0], dtype=jnp.int32)
    for k in range(TOPK):
        cnt = cnt + (cs_reg[k] <= np.float32(QHAT)).astype(jnp.int32)
    sizes_base = cnt + 1

    idx = sizes_base - 1
    ord_at = jnp.zeros_like(s[0])
    cs_at = jnp.zeros_like(s[0])
    for k in range(TOPK):
        sel = idx == k
        ord_at = jnp.where(sel, ord_reg[k], ord_at)
        cs_at = jnp.where(sel, cs_reg[k], cs_at)
    v = (cs_at - np.float32(QHAT)) / ord_at

    u = u_ref[...].reshape(ROWS, 1)
    sizes = sizes_base - (u <= v).astype(jnp.int32)

    cutoff = jnp.full_like(s[0], jnp.inf)
    for k in range(TOPK):
        cutoff = jnp.where(sizes - 1 == k, tops[k], cutoff)

    def p3(c, carry):
        x_c = x_ref[:, pl.ds(c * LW, LW)]
        mask_ref[:, pl.ds(c * LW, LW)] = x_c >= cutoff
        return carry

    lax.fori_loop(0, nfull, p3, 0)
    if tail_w:
        mask_ref[:, nfull * LW:n] = x_ref[:, nfull * LW:n] >= cutoff
    sizes_ref[...] = sizes.reshape(1, 1, ROWS)


def kernel(logits):
    b, n = logits.shape
    g = b // ROWS
    u = jax.random.uniform(jax.random.key(1), (b,), dtype=logits.dtype)
    u3 = u.reshape(g, 1, ROWS)

    mask, sizes3 = pl.pallas_call(
        _body,
        grid=(g,),
        in_specs=[
            pl.BlockSpec((ROWS, n), lambda i: (i, 0)),
            pl.BlockSpec((1, 1, ROWS), lambda i: (i, 0, 0)),
        ],
        out_specs=[
            pl.BlockSpec((ROWS, n), lambda i: (i, 0)),
            pl.BlockSpec((1, 1, ROWS), lambda i: (i, 0, 0)),
        ],
        out_shape=[
            jax.ShapeDtypeStruct((b, n), jnp.bool_),
            jax.ShapeDtypeStruct((g, 1, ROWS), jnp.int32),
        ],
    )(logits, u3)

    return (logits, sizes3.reshape(b), mask)
